# baseline (speedup 1.0000x reference)
import jax
import jax.numpy as jnp
from jax.experimental import pallas as pl
from jax.experimental.pallas import tpu as pltpu

_HIDDEN = 1024
_E = 16
_FFN = 4096
_NTOK = 256
_FCHUNK = 1024
_NF = _FFN // _FCHUNK


def _moe_body(x_ref, wr_ref, w1_ref, b1_ref, w2_ref, b2_ref, out_ref, wts_ref):
    e = pl.program_id(0)
    f = pl.program_id(1)

    lane = jax.lax.broadcasted_iota(jnp.int32, (_NTOK, _E), 1)

    @pl.when((e == 0) & (f == 0))
    def _router():
        logits = jax.lax.dot_general(
            x_ref[...], wr_ref[...], (((1,), (1,)), ((), ())),
            preferred_element_type=jnp.float32,
        )
        m1 = jnp.max(logits, axis=1, keepdims=True)
        i1 = jnp.min(jnp.where(logits == m1, lane, _E), axis=1, keepdims=True)
        masked = jnp.where(lane == i1, -jnp.inf, logits)
        m2 = jnp.max(masked, axis=1, keepdims=True)
        i2 = jnp.min(jnp.where(masked == m2, lane, _E), axis=1, keepdims=True)
        t = jnp.exp(m2 - m1)
        p1 = 1.0 / (1.0 + t)
        p2 = t / (1.0 + t)
        wts_ref[...] = jnp.where(lane == i1, p1, 0.0) + jnp.where(lane == i2, p2, 0.0)

    xb = x_ref[...].astype(jnp.bfloat16)
    h = jax.lax.dot_general(
        xb, w1_ref[0].astype(jnp.bfloat16), (((1,), (1,)), ((), ())),
        preferred_element_type=jnp.float32,
    )
    h = h + b1_ref[0]
    a = 0.5 * h * (1.0 + jax.lax.erf(h * 0.7071067811865476))
    ab = a.astype(jnp.bfloat16)
    wcol = jnp.sum(wts_ref[...] * (lane == e).astype(jnp.float32),
                   axis=1, keepdims=True)

    for fi in range(_NF):
        @pl.when(f == fi)
        def _contract(fi=fi):
            w2c = w2_ref[0, :, fi * _FCHUNK:(fi + 1) * _FCHUNK].astype(jnp.bfloat16)
            o = jax.lax.dot_general(
                ab, w2c, (((1,), (1,)), ((), ())),
                preferred_element_type=jnp.float32,
            )
            o = jnp.where(f == 0, o + b2_ref[0], o)
            contrib = wcol * o

            @pl.when((e == 0) & (f == 0))
            def _init():
                out_ref[...] = contrib

            @pl.when(~((e == 0) & (f == 0)))
            def _acc():
                out_ref[...] += contrib


def kernel(x, Wr, W1, b1, W2, b2):
    B, S, D = x.shape
    xf = x.reshape(B * S, D)
    b1r = b1.reshape(_E * _NF, 1, _FCHUNK)
    b2r = b2.reshape(_E, 1, _HIDDEN)
    out = pl.pallas_call(
        _moe_body,
        grid=(_E, _NF),
        in_specs=[
            pl.BlockSpec((_NTOK, _HIDDEN), lambda e, f: (0, 0)),
            pl.BlockSpec((_E, _HIDDEN), lambda e, f: (0, 0)),
            pl.BlockSpec((1, _FCHUNK, _HIDDEN), lambda e, f: (e, f, 0)),
            pl.BlockSpec((1, 1, _FCHUNK), lambda e, f: (e * _NF + f, 0, 0)),
            pl.BlockSpec((1, _HIDDEN, _FFN), lambda e, f: (e, 0, 0)),
            pl.BlockSpec((1, 1, _HIDDEN), lambda e, f: (e, 0, 0)),
        ],
        out_specs=pl.BlockSpec((_NTOK, _HIDDEN), lambda e, f: (0, 0)),
        out_shape=jax.ShapeDtypeStruct((_NTOK, _HIDDEN), jnp.float32),
        scratch_shapes=[pltpu.VMEM((_NTOK, _E), jnp.float32)],
        compiler_params=pltpu.CompilerParams(
            dimension_semantics=("arbitrary", "arbitrary"),
        ),
    )(xf, Wr, W1, b1r, W2, b2r)
    return out.reshape(B, S, D)


# speedup vs baseline: 1.2757x; 1.2757x over previous
import jax
import jax.numpy as jnp
from jax.experimental import pallas as pl
from jax.experimental.pallas import tpu as pltpu

_HIDDEN = 1024
_E = 16
_FFN = 4096
_NTOK = 256
_FC = 2048
_HC = 512
_NF = 2


def _w2_chunk_idx(e, f):
    ep = jnp.maximum(e - 1, 0)
    r = jnp.where((e % 2) == 1, 1 - f, f)
    r = jnp.where(e == 0, 1, r)
    return ep, r


def _moe_body(x_ref, wr_ref, w1_ref, b1_ref, w2_ref, b2_ref, out_ref,
              wts_ref, a0_ref, a1_ref):
    e = pl.program_id(0)
    f = pl.program_id(1)

    lane = jax.lax.broadcasted_iota(jnp.int32, (_NTOK, _E), 1)

    @pl.when((e == 0) & (f == 0))
    def _router():
        logits = jax.lax.dot_general(
            x_ref[...], wr_ref[...], (((1,), (1,)), ((), ())),
            preferred_element_type=jnp.float32,
        )
        m1 = jnp.max(logits, axis=1, keepdims=True)
        i1 = jnp.min(jnp.where(logits == m1, lane, _E), axis=1, keepdims=True)
        masked = jnp.where(lane == i1, -jnp.inf, logits)
        m2 = jnp.max(masked, axis=1, keepdims=True)
        i2 = jnp.min(jnp.where(masked == m2, lane, _E), axis=1, keepdims=True)
        t = jnp.exp(m2 - m1)
        p1 = 1.0 / (1.0 + t)
        p2 = t / (1.0 + t)
        wts_ref[...] = jnp.where(lane == i1, p1, 0.0) + jnp.where(lane == i2, p2, 0.0)

    @pl.when(e < _E)
    def _stage1():
        xb = x_ref[...].astype(jnp.bfloat16)
        h = jax.lax.dot_general(
            xb, w1_ref[0].astype(jnp.bfloat16), (((1,), (1,)), ((), ())),
            preferred_element_type=jnp.float32,
        )
        h = h + b1_ref[0]
        a = 0.5 * h * (1.0 + jax.lax.erf(h * 0.7071067811865476))
        ab = a.astype(jnp.bfloat16)
        for pi in range(2):
            for fi in range(_NF):
                @pl.when(((e % 2) == pi) & (f == fi))
                def _store(pi=pi, fi=fi, ab=ab):
                    dst = a0_ref if pi == 0 else a1_ref
                    dst[:, fi * _FC:(fi + 1) * _FC] = ab

    @pl.when(e >= 1)
    def _stage2():
        ep = e - 1
        wcol = jnp.sum(wts_ref[...] * (lane == ep).astype(jnp.float32),
                       axis=1, keepdims=True)
        w2b = w2_ref[0].astype(jnp.bfloat16)
        for pi in range(2):
            @pl.when((ep % 2) == pi)
            def _contract(pi=pi):
                src = a0_ref if pi == 0 else a1_ref
                o = jax.lax.dot_general(
                    src[...], w2b, (((1,), (1,)), ((), ())),
                    preferred_element_type=jnp.float32,
                )
                o = o + b2_ref[0]
                contrib = wcol * o
                r = jnp.where((e % 2) == 1, 1 - f, f)
                for ri in range(2):
                    @pl.when(r == ri)
                    def _acc(ri=ri, contrib=contrib):
                        @pl.when(e == 1)
                        def _init():
                            out_ref[:, ri * _HC:(ri + 1) * _HC] = contrib

                        @pl.when(e > 1)
                        def _add():
                            out_ref[:, ri * _HC:(ri + 1) * _HC] += contrib


def kernel(x, Wr, W1, b1, W2, b2):
    B, S, D = x.shape
    xf = x.reshape(B * S, D)
    b1r = b1.reshape(_E * _NF, 1, _FC)
    b2r = b2.reshape(_E * 2, 1, _HC)

    def w1_map(e, f):
        ec = jnp.minimum(e, _E - 1)
        fc = jnp.where(e == _E, 1, f)
        return (ec, fc, 0)

    def w2_map(e, f):
        ep, r = _w2_chunk_idx(e, f)
        return (ep, r, 0)

    def b1_map(e, f):
        ec = jnp.minimum(e, _E - 1)
        fc = jnp.where(e == _E, 1, f)
        return (ec * _NF + fc, 0, 0)

    def b2_map(e, f):
        ep, r = _w2_chunk_idx(e, f)
        return (ep * 2 + r, 0, 0)

    out = pl.pallas_call(
        _moe_body,
        grid=(_E + 1, _NF),
        in_specs=[
            pl.BlockSpec((_NTOK, _HIDDEN), lambda e, f: (0, 0)),
            pl.BlockSpec((_E, _HIDDEN), lambda e, f: (0, 0)),
            pl.BlockSpec((1, _FC, _HIDDEN), w1_map),
            pl.BlockSpec((1, 1, _FC), b1_map),
            pl.BlockSpec((1, _HC, _FFN), w2_map),
            pl.BlockSpec((1, 1, _HC), b2_map),
        ],
        out_specs=pl.BlockSpec((_NTOK, _HIDDEN), lambda e, f: (0, 0)),
        out_shape=jax.ShapeDtypeStruct((_NTOK, _HIDDEN), jnp.float32),
        scratch_shapes=[
            pltpu.VMEM((_NTOK, _E), jnp.float32),
            pltpu.VMEM((_NTOK, _FFN), jnp.bfloat16),
            pltpu.VMEM((_NTOK, _FFN), jnp.bfloat16),
        ],
        compiler_params=pltpu.CompilerParams(
            dimension_semantics=("arbitrary", "arbitrary"),
        ),
    )(xf, Wr, W1, b1r, W2, b2r)
    return out.reshape(B, S, D)
